# baseline (speedup 1.0000x reference)
import numpy as np
import jax
import jax.numpy as jnp
from jax.experimental import pallas as pl
from jax.experimental.pallas import tpu as pltpu

_GRAD_THRESHOLD = 0.5
_MIN_OPACITY = 0.05
_LOG2 = float(np.log(2.0))

_WIDTHS = (3, 3, 4, 1, 3, 9, 2, 1)


def _norm_body(n, b, sc_ref, out_ref):
    i = pl.program_id(0)
    s = jnp.exp(jnp.transpose(sc_ref[...]))
    n2 = jnp.sum(s * s, axis=0, keepdims=True)
    col = i * b + jax.lax.broadcasted_iota(jnp.int32, (1, b), 1)
    norm = jnp.where(col < n, jnp.sqrt(n2), jnp.float32(np.inf))
    out_ref[...] = norm[None]


def _median_body(k1, k2, x_ref, thr_ref):
    x = x_ref[...]
    xi = jax.lax.bitcast_convert_type(x, jnp.int32)

    def cnt_le(t):
        return jnp.sum((xi <= t).astype(jnp.int32))

    def it(_, carry):
        lo, hi = carry
        mid = lo + (hi - lo) // 2
        pred = cnt_le(mid) >= k1
        lo2 = jnp.where(pred, lo, mid)
        hi2 = jnp.where(pred, mid, hi)
        return lo2, hi2

    lo0 = jnp.int32(-1)
    hi0 = jnp.int32(0x7F800000)
    _, a_int = jax.lax.fori_loop(0, 31, it, (lo0, hi0))
    neg_inf = jnp.float32(-np.inf)
    pos_inf = jnp.float32(np.inf)
    a = jnp.max(jnp.where(xi <= a_int, x, neg_inf))
    c_a = cnt_le(a_int)
    b = jnp.where(c_a >= k2, a, jnp.min(jnp.where(xi > a_int, x, pos_inf)))
    thr_ref[0, 0] = (a + b) * 0.5


def _main_body(n, bb, nbm, thr_ref, pos_h, sc_h, rot_h, op_h, dc_h, rest_h,
               ga_h, gc_h, sn_h, out_h, pos_v, sc_v, rot_v, op_v, dc_v,
               rest_v, ga_v, gc_v, sn_s, out_s, lsem, ssem):
    ins = (pos_v, sc_v, rot_v, op_v, dc_v, rest_v, ga_v, gc_v)
    i = pl.program_id(0)
    slot = jax.lax.rem(i, 2)
    hbm = (pos_h, sc_h, rot_h, op_h, dc_h, rest_h, ga_h, gc_h)

    def copies(step, buf):
        sl = pl.ds(step * bb, bb)
        cps = [pltpu.make_async_copy(h.at[sl, :], s.at[buf],
                                     lsem.at[buf, k])
               for k, (h, s) in enumerate(zip(hbm, ins))]
        cps.append(pltpu.make_async_copy(sn_h.at[:, sl, :], sn_s.at[buf],
                                         lsem.at[buf, 8]))
        return cps

    @pl.when(i == 0)
    def _():
        for c in copies(0, 0):
            c.start()

    @pl.when(i + 1 < nbm)
    def _():
        for c in copies(i + 1, jax.lax.rem(i + 1, 2)):
            c.start()

    for c in copies(i, slot):
        c.wait()

    pos_s, sc_s, rot_s, op_s, dc_s, rest_s, ga_s, gc_s = ins
    t = jnp.transpose
    pos = t(pos_s[slot])
    sc = t(sc_s[slot])
    ga = t(ga_s[slot])
    gcf = t(gc_s[slot].astype(jnp.float32))
    opac = t(op_s[slot])

    thr = thr_ref[0, 0]
    cnts = jnp.maximum(gcf, 1.0)
    avg = ga / cnts
    gn2 = jnp.sum(avg * avg, axis=0, keepdims=True)
    large = gn2 >= _GRAD_THRESHOLD * _GRAD_THRESHOLD
    asc = jnp.exp(sc)
    snorm = jnp.sqrt(jnp.sum(asc * asc, axis=0, keepdims=True))
    clone = large & (snorm <= thr)
    split = large & (snorm > thr)
    act_op = jax.nn.sigmoid(opac)
    keep = jnp.logical_not((act_op < _MIN_OPACITY) | split)

    one = jnp.float32(1.0)
    zero = jnp.float32(0.0)
    kf = jnp.where(keep, one, zero)
    cf = jnp.where(clone, one, zero)
    sf = jnp.where(split, one, zero)

    p = jnp.concatenate(
        [pos, sc, t(rot_s[slot]), opac, t(dc_s[slot]), t(rest_s[slot])],
        axis=0)

    @pl.when(i >= 2)
    def _():
        pltpu.make_async_copy(
            out_s.at[slot], out_h.at[:, pl.ds((i - 2) * bb, bb), :],
            ssem.at[slot]).wait()

    out_s[slot, 0] = t(p * kf)
    out_s[slot, 1] = t(p * cf)
    sp_sc = sc - _LOG2
    tail = p[6:23]
    for u in range(2):
        sn = t(sn_s[slot, u])
        pi = jnp.concatenate([pos + sn * asc, sp_sc, tail], axis=0)
        out_s[slot, 2 + u] = t(pi * sf)

    pltpu.make_async_copy(out_s.at[slot],
                          out_h.at[:, pl.ds(i * bb, bb), :],
                          ssem.at[slot]).start()

    @pl.when(i == nbm - 1)
    def _():
        if nbm >= 2:
            pltpu.make_async_copy(
                out_s.at[1 - slot], out_h.at[:, pl.ds((i - 1) * bb, bb), :],
                ssem.at[1 - slot]).wait()
        pltpu.make_async_copy(out_s.at[slot],
                              out_h.at[:, pl.ds(i * bb, bb), :],
                              ssem.at[slot]).wait()


def _pick_div(total, cap):
    best = 1
    for d in range(1, cap + 1):
        if total % d == 0:
            best = d
    return best


def _build(n, interpret=False):
    f32 = jnp.float32
    b = 25600 if n >= 25600 else ((n + 7) // 8) * 8
    nb = -(-n // b)
    npad = nb * b

    norms_call = pl.pallas_call(
        lambda sc_ref, out_ref: _norm_body(n, b, sc_ref, out_ref),
        grid=(nb,),
        in_specs=[pl.BlockSpec((b, 3), lambda i: (i, 0))],
        out_specs=pl.BlockSpec((1, 1, b), lambda i: (i, 0, 0)),
        out_shape=jax.ShapeDtypeStruct((nb, 1, b), f32),
        interpret=interpret,
    )

    k1 = n // 2
    k2 = n // 2 + 1
    median_call = pl.pallas_call(
        lambda x_ref, t_ref: _median_body(k1, k2, x_ref, t_ref),
        in_specs=[pl.BlockSpec(memory_space=pltpu.VMEM)],
        out_specs=pl.BlockSpec(memory_space=pltpu.SMEM),
        out_shape=jax.ShapeDtypeStruct((1, 1), f32),
        interpret=interpret,
    )

    bb = _pick_div(n, 2500)
    nbm = n // bb
    widths = _WIDTHS
    main_call = pl.pallas_call(
        lambda *a: _main_body(n, bb, nbm, *a),
        grid=(nbm,),
        in_specs=[pl.BlockSpec(memory_space=pltpu.SMEM)]
        + [pl.BlockSpec(memory_space=pl.ANY)] * 9,
        out_specs=pl.BlockSpec(memory_space=pl.ANY),
        out_shape=jax.ShapeDtypeStruct((4, n, 23), f32),
        scratch_shapes=(
            [pltpu.VMEM((2, bb, w), jnp.int32 if k == 7 else f32)
             for k, w in enumerate(widths)]
            + [pltpu.VMEM((2, 2, bb, 3), f32),
               pltpu.VMEM((2, 4, bb, 23), f32),
               pltpu.SemaphoreType.DMA((2, 9)),
               pltpu.SemaphoreType.DMA((2,))]
        ),
        interpret=interpret,
    )

    def run(positions, scales, rotations, opacities, sh_dc, sh_rest,
            grad_accum, grad_count, split_noise):
        norms = norms_call(scales)
        thr = median_call(norms.reshape(8, npad // 8))
        out4 = main_call(thr, positions, scales, rotations, opacities,
                         sh_dc, sh_rest, grad_accum,
                         grad_count.reshape(n, 1), split_noise)
        return out4.reshape(4 * n, 23)

    return run


_CACHE = {}


def kernel(positions, scales, rotations, opacities, sh_dc, sh_rest,
           grad_accum, grad_count, split_noise):
    n = positions.shape[0]
    if n not in _CACHE:
        _CACHE[n] = _build(n)
    return _CACHE[n](positions, scales, rotations, opacities, sh_dc, sh_rest,
                     grad_accum, grad_count, split_noise)


# speedup vs baseline: 1.0079x; 1.0079x over previous
import numpy as np
import jax
import jax.numpy as jnp
from jax.experimental import pallas as pl
from jax.experimental.pallas import tpu as pltpu

_GRAD_THRESHOLD = 0.5
_MIN_OPACITY = 0.05
_LOG2 = float(np.log(2.0))

_WIDTHS = (3, 3, 4, 1, 3, 9, 2, 1)


def _norm_body(n, b, sc_ref, out_ref):
    i = pl.program_id(0)
    s = jnp.exp(jnp.transpose(sc_ref[...]))
    n2 = jnp.sum(s * s, axis=0, keepdims=True)
    col = i * b + jax.lax.broadcasted_iota(jnp.int32, (1, b), 1)
    norm = jnp.where(col < n, jnp.sqrt(n2), jnp.float32(np.inf))
    out_ref[...] = norm[None]


def _median_body(k1, k2, x_ref, thr_ref):
    x = x_ref[...]
    xi = jax.lax.bitcast_convert_type(x, jnp.int32)

    def cnt_le(t):
        return jnp.sum((xi <= t).astype(jnp.int32))

    def it(_, carry):
        lo, hi = carry
        mid = lo + (hi - lo) // 2
        pred = cnt_le(mid) >= k1
        lo2 = jnp.where(pred, lo, mid)
        hi2 = jnp.where(pred, mid, hi)
        return lo2, hi2

    lo0 = jnp.int32(-1)
    hi0 = jnp.int32(0x7F800000)
    _, a_int = jax.lax.fori_loop(0, 31, it, (lo0, hi0))
    neg_inf = jnp.float32(-np.inf)
    pos_inf = jnp.float32(np.inf)
    a = jnp.max(jnp.where(xi <= a_int, x, neg_inf))
    c_a = cnt_le(a_int)
    b = jnp.where(c_a >= k2, a, jnp.min(jnp.where(xi > a_int, x, pos_inf)))
    thr_ref[0, 0] = (a + b) * 0.5


def _main_body(n, bb, nbm, thr_ref, pos_h, sc_h, rot_h, op_h, dc_h, rest_h,
               ga_h, gc_h, sn_h, out_h, pos_v, sc_v, rot_v, op_v, dc_v,
               rest_v, ga_v, gc_v, sn_s, out_s, lsem, ssem):
    ins = (pos_v, sc_v, rot_v, op_v, dc_v, rest_v, ga_v, gc_v)
    i = pl.program_id(0)
    slot = jax.lax.rem(i, 2)
    hbm = (pos_h, sc_h, rot_h, op_h, dc_h, rest_h, ga_h, gc_h)

    def copies(step, buf):
        sl = pl.ds(step * bb, bb)
        cps = [pltpu.make_async_copy(h.at[sl, :], s.at[buf],
                                     lsem.at[buf, k])
               for k, (h, s) in enumerate(zip(hbm, ins))]
        cps.append(pltpu.make_async_copy(sn_h.at[:, sl, :], sn_s.at[buf],
                                         lsem.at[buf, 8]))
        return cps

    @pl.when(i == 0)
    def _():
        for c in copies(0, 0):
            c.start()

    @pl.when(i + 1 < nbm)
    def _():
        for c in copies(i + 1, jax.lax.rem(i + 1, 2)):
            c.start()

    for c in copies(i, slot):
        c.wait()

    pos_s, sc_s, rot_s, op_s, dc_s, rest_s, ga_s, gc_s = ins
    t = jnp.transpose
    pos = t(pos_s[slot])
    sc = t(sc_s[slot])
    ga = t(ga_s[slot])
    gcf = t(gc_s[slot].astype(jnp.float32))
    opac = t(op_s[slot])

    thr = thr_ref[0, 0]
    cnts = jnp.maximum(gcf, 1.0)
    avg = ga / cnts
    gn2 = jnp.sum(avg * avg, axis=0, keepdims=True)
    large = gn2 >= _GRAD_THRESHOLD * _GRAD_THRESHOLD
    asc = jnp.exp(sc)
    snorm = jnp.sqrt(jnp.sum(asc * asc, axis=0, keepdims=True))
    clone = large & (snorm <= thr)
    split = large & (snorm > thr)
    act_op = jax.nn.sigmoid(opac)
    keep = jnp.logical_not((act_op < _MIN_OPACITY) | split)

    one = jnp.float32(1.0)
    zero = jnp.float32(0.0)
    kf = jnp.where(keep, one, zero)
    cf = jnp.where(clone, one, zero)
    sf = jnp.where(split, one, zero)

    p = jnp.concatenate(
        [pos, sc, t(rot_s[slot]), opac, t(dc_s[slot]), t(rest_s[slot])],
        axis=0)

    @pl.when(i >= 2)
    def _():
        pltpu.make_async_copy(
            out_s.at[slot], out_h.at[:, pl.ds((i - 2) * bb, bb), :],
            ssem.at[slot]).wait()

    out_s[slot, 0] = t(p * kf)
    out_s[slot, 1] = t(p * cf)
    sp_sc = sc - _LOG2
    tail = p[6:23]
    for u in range(2):
        sn = t(sn_s[slot, u])
        pi = jnp.concatenate([pos + sn * asc, sp_sc, tail], axis=0)
        out_s[slot, 2 + u] = t(pi * sf)

    pltpu.make_async_copy(out_s.at[slot],
                          out_h.at[:, pl.ds(i * bb, bb), :],
                          ssem.at[slot]).start()

    @pl.when(i == nbm - 1)
    def _():
        if nbm >= 2:
            pltpu.make_async_copy(
                out_s.at[1 - slot], out_h.at[:, pl.ds((i - 1) * bb, bb), :],
                ssem.at[1 - slot]).wait()
        pltpu.make_async_copy(out_s.at[slot],
                              out_h.at[:, pl.ds(i * bb, bb), :],
                              ssem.at[slot]).wait()


def _pick_div(total, cap):
    best = 1
    for d in range(1, cap + 1):
        if total % d == 0:
            best = d
    return best


def _build(n, interpret=False):
    f32 = jnp.float32
    b = 25600 if n >= 25600 else ((n + 7) // 8) * 8
    nb = -(-n // b)
    npad = nb * b

    norms_call = pl.pallas_call(
        lambda sc_ref, out_ref: _norm_body(n, b, sc_ref, out_ref),
        grid=(nb,),
        in_specs=[pl.BlockSpec((b, 3), lambda i: (i, 0))],
        out_specs=pl.BlockSpec((1, 1, b), lambda i: (i, 0, 0)),
        out_shape=jax.ShapeDtypeStruct((nb, 1, b), f32),
        interpret=interpret,
    )

    k1 = n // 2
    k2 = n // 2 + 1
    median_call = pl.pallas_call(
        lambda x_ref, t_ref: _median_body(k1, k2, x_ref, t_ref),
        in_specs=[pl.BlockSpec(memory_space=pltpu.VMEM)],
        out_specs=pl.BlockSpec(memory_space=pltpu.SMEM),
        out_shape=jax.ShapeDtypeStruct((1, 1), f32),
        interpret=interpret,
    )

    bb = _pick_div(n, 3200)
    nbm = n // bb
    widths = _WIDTHS
    main_call = pl.pallas_call(
        lambda *a: _main_body(n, bb, nbm, *a),
        grid=(nbm,),
        in_specs=[pl.BlockSpec(memory_space=pltpu.SMEM)]
        + [pl.BlockSpec(memory_space=pl.ANY)] * 9,
        out_specs=pl.BlockSpec(memory_space=pl.ANY),
        out_shape=jax.ShapeDtypeStruct((4, n, 23), f32),
        scratch_shapes=(
            [pltpu.VMEM((2, bb, w), jnp.int32 if k == 7 else f32)
             for k, w in enumerate(widths)]
            + [pltpu.VMEM((2, 2, bb, 3), f32),
               pltpu.VMEM((2, 4, bb, 23), f32),
               pltpu.SemaphoreType.DMA((2, 9)),
               pltpu.SemaphoreType.DMA((2,))]
        ),
        interpret=interpret,
    )

    def run(positions, scales, rotations, opacities, sh_dc, sh_rest,
            grad_accum, grad_count, split_noise):
        norms = norms_call(scales)
        thr = median_call(norms.reshape(8, npad // 8))
        out4 = main_call(thr, positions, scales, rotations, opacities,
                         sh_dc, sh_rest, grad_accum,
                         grad_count.reshape(n, 1), split_noise)
        return out4.reshape(4 * n, 23)

    return run


_CACHE = {}


def kernel(positions, scales, rotations, opacities, sh_dc, sh_rest,
           grad_accum, grad_count, split_noise):
    n = positions.shape[0]
    if n not in _CACHE:
        _CACHE[n] = _build(n)
    return _CACHE[n](positions, scales, rotations, opacities, sh_dc, sh_rest,
                     grad_accum, grad_count, split_noise)
